# baseline (speedup 1.0000x reference)
import functools

import jax
import jax.numpy as jnp
from jax import lax
from jax.experimental import pallas as pl
from jax.experimental.pallas import tpu as pltpu
from jax.experimental.pallas import tpu_sc as plsc

_B, _S, _D = 4, 8192, 768
_N = _B * _S

_NW = 32
_TPW = _N // _NW
_CHUNK = 64
_NCH = _TPW // _CHUNK
_CSZ = _CHUNK * _D


def _sc_body(x_hbm, w_hbm, gb_hbm, hard_hbm, cnt_hbm,
             xb0, xb1, wrv, gbv, hv, cv, sem0, sem1):
    cid = lax.axis_index("c")
    sid = lax.axis_index("s")
    wid = sid * 2 + cid
    base = wid * _TPW

    pltpu.sync_copy(w_hbm, wrv)
    pltpu.sync_copy(gb_hbm.at[pl.ds(base, _TPW)], gbv)

    def chunk_slice(c):
        return x_hbm.at[pl.ds((base + c * _CHUNK) * _D, _CSZ)]

    def compute(xb, c, cnt):
        nb = _CHUNK // 16
        lane = lax.iota(jnp.int32, 16)
        rots = [(lane + k) & 15 for k in range(16)]
        idxs0 = [(tb * 16 + lax.iota(jnp.int32, 16)) * _D for tb in range(nb)]
        zero = jnp.zeros((16,), jnp.float32)

        def jbody(jj, accs):
            ae = list(accs[:nb])
            ao = list(accs[nb:])
            wb = jj * 16
            ibs = [idxs0[bi] + wb for bi in range(nb)]
            for k in range(0, 16, 2):
                we = plsc.load_gather(wrv, [wb + rots[k]])
                wo = plsc.load_gather(wrv, [wb + rots[k + 1]])
                for bi in range(nb):
                    xe = plsc.load_gather(xb, [ibs[bi] + rots[k]])
                    xo = plsc.load_gather(xb, [ibs[bi] + rots[k + 1]])
                    ue = plsc.bitcast(xe, jnp.int32)
                    uo = plsc.bitcast(xo, jnp.int32)
                    re_ = plsc.bitcast((ue + 0x8000) & -65536, jnp.float32)
                    ro_ = plsc.bitcast((uo + 0x8000) & -65536, jnp.float32)
                    ae[bi] = ae[bi] + re_ * we
                    ao[bi] = ao[bi] + ro_ * wo
            return tuple(ae) + tuple(ao)

        accs = lax.fori_loop(0, _D // 16, jbody, (zero,) * (2 * nb))
        for bi in range(nb):
            pos = c * _CHUNK + bi * 16
            xg = accs[bi] + accs[nb + bi] + gbv[pl.ds(pos, 16)]
            hard = jnp.where(xg >= 0.0, 1.0, 0.0).astype(jnp.float32)
            hv[pl.ds(pos, 16)] = hard
            cnt = cnt + hard
        return cnt

    pltpu.async_copy(chunk_slice(0), xb0, sem0)

    def sbody(s, cnt):
        c0 = s * 2
        pltpu.async_copy(chunk_slice(c0 + 1), xb1, sem1)
        pltpu.make_async_copy(chunk_slice(c0), xb0, sem0).wait()
        cnt = compute(xb0, c0, cnt)

        @pl.when(s < _NCH // 2 - 1)
        def _next():
            pltpu.async_copy(chunk_slice(c0 + 2), xb0, sem0)

        pltpu.make_async_copy(chunk_slice(c0 + 1), xb1, sem1).wait()
        cnt = compute(xb1, c0 + 1, cnt)
        return cnt

    cnt = lax.fori_loop(0, _NCH // 2, sbody, jnp.zeros((16,), jnp.float32))

    cv[...] = cnt
    pltpu.sync_copy(hv, hard_hbm.at[pl.ds(base, _TPW)])
    pltpu.sync_copy(cv, cnt_hbm.at[wid])


_sc_call = pl.kernel(
    _sc_body,
    out_type=[
        jax.ShapeDtypeStruct((_N,), jnp.float32),
        jax.ShapeDtypeStruct((_NW, 16), jnp.float32),
    ],
    mesh=plsc.VectorSubcoreMesh(
        core_axis_name="c", subcore_axis_name="s",
        num_cores=2, num_subcores=16,
    ),
    compiler_params=pltpu.CompilerParams(needs_layout_passes=False),
    scratch_types=[
        pltpu.VMEM((_CSZ,), jnp.float32),
        pltpu.VMEM((_CSZ,), jnp.float32),
        pltpu.VMEM((_D,), jnp.float32),
        pltpu.VMEM((_TPW,), jnp.float32),
        pltpu.VMEM((_TPW,), jnp.float32),
        pltpu.VMEM((16,), jnp.float32),
        pltpu.SemaphoreType.DMA,
        pltpu.SemaphoreType.DMA,
    ],
)


def _noise_plus_bias(b):
    eps = 1e-08
    nk1, nk2 = jax.random.split(jax.random.key(1))
    u1 = jax.random.uniform(nk1, (_N,), dtype=jnp.float32)
    u2 = jax.random.uniform(nk2, (_N,), dtype=jnp.float32)
    g1 = -jnp.log(-jnp.log(u1 + eps) + eps)
    g2 = -jnp.log(-jnp.log(u2 + eps) + eps)
    return g1 - g2 + b[0]


@jax.jit
def kernel(x, W, b):
    x1 = x.reshape(_N * _D)
    wi = jax.lax.bitcast_convert_type(W.reshape(_D), jnp.int32)
    wi = (wi + 0x7FFF + ((wi >> 16) & 1)) & -65536
    wr = jax.lax.bitcast_convert_type(wi, jnp.float32)
    gb = _noise_plus_bias(b)
    hard, cnt = _sc_call(x1, wr, gb)
    maskloss = (jnp.sum(cnt) / _N).astype(jnp.float32)
    return hard.reshape(_B, _S, 1), maskloss


# speedup vs baseline: 4.4267x; 4.4267x over previous
import functools

import jax
import jax.numpy as jnp
from jax import lax
from jax.experimental import pallas as pl
from jax.experimental.pallas import tpu as pltpu

_B, _S, _D = 4, 8192, 768
_N = _B * _S
_BLK = 4096
_ROWS = _BLK // 128


def _mask_kernel(x_ref, w_ref, gb_ref, hard_ref, cnt_ref):
    i = pl.program_id(0)
    xb = x_ref[...].astype(jnp.bfloat16).astype(jnp.float32)
    wb = w_ref[...].astype(jnp.bfloat16).astype(jnp.float32).reshape(1, _D)
    s = jnp.sum(xb * wb, axis=1).reshape(_ROWS, 128)
    xg = s + gb_ref[...]
    hard = (xg >= 0.0).astype(jnp.float32)
    hard_ref[...] = hard

    @pl.when(i == 0)
    def _init():
        cnt_ref[...] = jnp.zeros_like(cnt_ref)

    cnt_ref[...] += jnp.sum(hard, axis=0, keepdims=True).reshape(1, 128)


def _noise_plus_bias(b):
    eps = 1e-08
    nk1, nk2 = jax.random.split(jax.random.key(1))
    u1 = jax.random.uniform(nk1, (_N // 128, 128), dtype=jnp.float32)
    u2 = jax.random.uniform(nk2, (_N // 128, 128), dtype=jnp.float32)
    g1 = -jnp.log(-jnp.log(u1 + eps) + eps)
    g2 = -jnp.log(-jnp.log(u2 + eps) + eps)
    return g1 - g2 + b[0]


@jax.jit
def kernel(x, W, b):
    x2 = x.reshape(_N, _D)
    gb = _noise_plus_bias(b)
    hard, cnt = pl.pallas_call(
        _mask_kernel,
        grid=(_N // _BLK,),
        in_specs=[
            pl.BlockSpec((_BLK, _D), lambda i: (i, 0)),
            pl.BlockSpec((_D, 1), lambda i: (0, 0)),
            pl.BlockSpec((_ROWS, 128), lambda i: (i, 0)),
        ],
        out_specs=[
            pl.BlockSpec((_ROWS, 128), lambda i: (i, 0)),
            pl.BlockSpec((1, 128), lambda i: (0, 0)),
        ],
        out_shape=[
            jax.ShapeDtypeStruct((_N // 128, 128), jnp.float32),
            jax.ShapeDtypeStruct((1, 128), jnp.float32),
        ],
    )(x2, W, gb)
    maskloss = (jnp.sum(cnt) / _N).astype(jnp.float32)
    return hard.reshape(_B, _S, 1), maskloss
